# Initial kernel scaffold; baseline (speedup 1.0000x reference)
import jax
import jax.numpy as jnp
from jax.experimental import pallas as pl


def kernel(x, W):
    raise NotImplementedError("write your pallas kernel here")


# speedup vs baseline: 3.7049x; 3.7049x over previous
import functools

import jax
import jax.numpy as jnp
from jax import lax
from jax.experimental import pallas as pl
from jax.experimental.pallas import tpu as pltpu
from jax.experimental.pallas import tpu_sc as plsc

_CHUNK = 64


@functools.partial(jax.jit, static_argnums=(1, 2))
def _sc_broadcast_rows(W, batch, seq_len):
    d = W.shape[1]
    info = plsc.get_sparse_core_info()
    nw = info.num_cores * info.num_subcores
    rows_per_w = seq_len // nw
    chunk = min(_CHUNK, rows_per_w)
    n_chunks = rows_per_w // chunk
    mesh = plsc.VectorSubcoreMesh(core_axis_name="c", subcore_axis_name="s")

    @functools.partial(
        pl.kernel,
        mesh=mesh,
        out_type=jax.ShapeDtypeStruct((batch * seq_len, d), jnp.float32),
        scratch_types=[
            pltpu.VMEM((chunk, d), jnp.float32),
            pltpu.VMEM((chunk, d), jnp.float32),
            pltpu.SemaphoreType.DMA,
            pltpu.SemaphoreType.DMA,
        ],
    )
    def k(w_hbm, out_hbm, buf0, buf1, in_sem, out_sem):
        wid = lax.axis_index("s") * info.num_cores + lax.axis_index("c")
        base = wid * rows_per_w
        bufs = (buf0, buf1)

        pltpu.async_copy(w_hbm.at[pl.ds(base, chunk)], buf0, in_sem)

        for i in range(n_chunks):
            cur = bufs[i % 2]
            pltpu.make_async_copy(
                w_hbm.at[pl.ds(base + i * chunk, chunk)], cur, in_sem
            ).wait()
            if i + 1 < n_chunks:
                pltpu.async_copy(
                    w_hbm.at[pl.ds(base + (i + 1) * chunk, chunk)],
                    bufs[(i + 1) % 2],
                    in_sem,
                )
            r0 = base + i * chunk
            for b in range(batch):
                pltpu.async_copy(
                    cur, out_hbm.at[pl.ds(b * seq_len + r0, chunk)], out_sem
                )
            for b in range(batch):
                pltpu.make_async_copy(
                    cur, out_hbm.at[pl.ds(b * seq_len + r0, chunk)], out_sem
                ).wait()

    return k(W)


def kernel(x, W):
    batch, seq_len = x.shape
    d = W.shape[1]
    flat = _sc_broadcast_rows(W, batch, seq_len)
    return flat.reshape(batch, seq_len, d)
